# Initial kernel scaffold; baseline (speedup 1.0000x reference)
import jax
import jax.numpy as jnp
from jax.experimental import pallas as pl


def kernel(x, user_tables, item_tables, uW1, ub1, uW2, ub2, uW3, ub3, iW1, ib1, iW2, ib2, iW3, ib3):
    raise NotImplementedError("write your pallas kernel here")


# speedup vs baseline: 1.2208x; 1.2208x over previous
import functools

import jax
import jax.numpy as jnp
from jax import lax
from jax.experimental import pallas as pl
from jax.experimental.pallas import tpu as pltpu
from jax.experimental.pallas import tpu_sc as plsc

B = 4096
VOCAB = 100000
EMB = 32
N_US = 13
N_UD = 5
N_IS = 13
N_ID = 5

NW = 32
ROWS = B * N_US
RPW = ROWS // NW
CHUNK = 128
NCH = RPW // CHUNK


def _gather_body(utab, itab, uidx, iidx, uout, iout,
                 uidx_v, urows_v, iidx_v, irows_v, usem, isem):
    nc = 2
    wid = lax.axis_index("s") * nc + lax.axis_index("c")

    pltpu.sync_copy(uidx.at[wid], uidx_v)
    pltpu.sync_copy(iidx.at[wid], iidx_v)

    lanes = lax.iota(jnp.int32, 16)
    for j in range(NCH):
        for k in range(CHUNK // 16):
            c = (128 * j + 16 * k) % N_US
            off = lax.rem(lanes + c, jnp.int32(N_US)) * jnp.int32(VOCAB)
            uidx_v[j, pl.ds(k * 16, 16)] = uidx_v[j, pl.ds(k * 16, 16)] + off
            iidx_v[j, pl.ds(k * 16, 16)] = iidx_v[j, pl.ds(k * 16, 16)] + off

    copies = []
    for j in range(NCH):
        copies.append(pltpu.async_copy(
            utab.at[uidx_v.at[j]], urows_v.at[pl.ds(j * CHUNK, CHUNK)], usem))
        copies.append(pltpu.async_copy(
            itab.at[iidx_v.at[j]], irows_v.at[pl.ds(j * CHUNK, CHUNK)], isem))
    for c in copies:
        c.wait()

    pltpu.sync_copy(urows_v, uout.at[pl.ds(wid * RPW, RPW)])
    pltpu.sync_copy(irows_v, iout.at[pl.ds(wid * RPW, RPW)])


@functools.cache
def _gather():
    return functools.partial(
        pl.kernel,
        mesh=plsc.VectorSubcoreMesh(core_axis_name="c", subcore_axis_name="s"),
        out_type=[jax.ShapeDtypeStruct((ROWS, EMB), jnp.float32),
                  jax.ShapeDtypeStruct((ROWS, EMB), jnp.float32)],
        scratch_types=[
            pltpu.VMEM((NCH, CHUNK), jnp.int32),
            pltpu.VMEM((RPW, EMB), jnp.float32),
            pltpu.VMEM((NCH, CHUNK), jnp.int32),
            pltpu.VMEM((RPW, EMB), jnp.float32),
            pltpu.SemaphoreType.DMA,
            pltpu.SemaphoreType.DMA,
        ],
        compiler_params=pltpu.CompilerParams(use_tc_tiling_on_sc=False),
    )(_gather_body)


BT = 256


def _mlp_body(ue_ref, ud_ref, ie_ref, id_ref,
              uW1e_ref, uW1d_ref, ub1_ref, uW2_ref, ub2_ref, uW3_ref, ub3_ref,
              iW1e_ref, iW1d_ref, ib1_ref, iW2_ref, ib2_ref, iW3_ref, ib3_ref,
              out_ref):
    def tower(e, d, W1e, W1d, b1, W2, b2, W3, b3):
        h = jnp.dot(e, W1e, preferred_element_type=jnp.float32)
        h = h + jnp.dot(d, W1d, preferred_element_type=jnp.float32) + b1
        h = jnp.maximum(h, 0.0)
        h = jnp.maximum(jnp.dot(h, W2, preferred_element_type=jnp.float32) + b2, 0.0)
        h = jnp.dot(h, W3, preferred_element_type=jnp.float32) + b3
        norm = jnp.sqrt(jnp.sum(h * h, axis=1, keepdims=True))
        return h / norm

    u = tower(ue_ref[...], ud_ref[...], uW1e_ref[...], uW1d_ref[...],
              ub1_ref[...], uW2_ref[...], ub2_ref[...], uW3_ref[...], ub3_ref[...])
    i = tower(ie_ref[...], id_ref[...], iW1e_ref[...], iW1d_ref[...],
              ib1_ref[...], iW2_ref[...], ib2_ref[...], iW3_ref[...], ib3_ref[...])
    out_ref[...] = jnp.concatenate([u, i], axis=1)


def _mlp(ue, ud, ie, idn, uW1e, uW1d, ub1, uW2, ub2, uW3, ub3,
         iW1e, iW1d, ib1, iW2, ib2, iW3, ib3, interpret=False):
    full = lambda shape: pl.BlockSpec(shape, lambda i: (0, 0))
    return pl.pallas_call(
        _mlp_body,
        grid=(B // BT,),
        in_specs=[
            pl.BlockSpec((BT, N_US * EMB), lambda i: (i, 0)),
            pl.BlockSpec((BT, N_UD), lambda i: (i, 0)),
            pl.BlockSpec((BT, N_IS * EMB), lambda i: (i, 0)),
            pl.BlockSpec((BT, N_ID), lambda i: (i, 0)),
            full(uW1e.shape), full(uW1d.shape), full(ub1.shape),
            full(uW2.shape), full(ub2.shape), full(uW3.shape), full(ub3.shape),
            full(iW1e.shape), full(iW1d.shape), full(ib1.shape),
            full(iW2.shape), full(ib2.shape), full(iW3.shape), full(ib3.shape),
        ],
        out_specs=pl.BlockSpec((BT, 128), lambda i: (i, 0)),
        out_shape=jax.ShapeDtypeStruct((B, 128), jnp.float32),
        compiler_params=pltpu.CompilerParams(
            dimension_semantics=("arbitrary",)),
        interpret=interpret,
    )(ue, ud, ie, idn, uW1e, uW1d, ub1, uW2, ub2, uW3, ub3,
      iW1e, iW1d, ib1, iW2, ib2, iW3, ib3)


def kernel(x, user_tables, item_tables, uW1, ub1, uW2, ub2, uW3, ub3,
           iW1, ib1, iW2, ib2, iW3, ib3):
    us = x[:, :N_US].astype(jnp.int32)
    ud = x[:, N_US:N_US + N_UD]
    it = x[:, N_US + N_UD:N_US + N_UD + N_IS].astype(jnp.int32)
    idn = x[:, N_US + N_UD + N_IS:]

    uidx = us.reshape(NW, NCH, CHUNK)
    iidx = it.reshape(NW, NCH, CHUNK)
    ue_flat, ie_flat = _gather()(
        user_tables.reshape(N_US * VOCAB, EMB),
        item_tables.reshape(N_IS * VOCAB, EMB),
        uidx, iidx)
    ue = ue_flat.reshape(B, N_US * EMB)
    ie = ie_flat.reshape(B, N_IS * EMB)

    D_E = N_US * EMB
    return _mlp(ue, ud, ie, idn,
                uW1[:D_E], uW1[D_E:], ub1.reshape(1, -1),
                uW2, ub2.reshape(1, -1), uW3, ub3.reshape(1, -1),
                iW1[:D_E], iW1[D_E:], ib1.reshape(1, -1),
                iW2, ib2.reshape(1, -1), iW3, ib3.reshape(1, -1))


# speedup vs baseline: 1.8641x; 1.5270x over previous
import functools

import jax
import jax.numpy as jnp
from jax import lax
from jax.experimental import pallas as pl
from jax.experimental.pallas import tpu as pltpu
from jax.experimental.pallas import tpu_sc as plsc

B = 4096
VOCAB = 100000
EMB = 32
N_US = 13
N_UD = 5
N_IS = 13
N_ID = 5

NW = 32
ROWS = B * N_US
RPW = ROWS // NW
CHUNK = 128
NCH = RPW // CHUNK

VB = 8192
VQ = VB // 4
VQ_SHIFT = VQ.bit_length() - 1
VPAD = -(-VOCAB // VB) * VB


def _gather_body(tab, idx, out, idx_v, rows_v, sem):
    nc = 2
    wid = lax.axis_index("s") * nc + lax.axis_index("c")

    pltpu.sync_copy(idx.at[wid], idx_v)

    lanes = lax.iota(jnp.int32, 16)
    for j in range(NCH):
        for k in range(CHUNK // 16):
            c = (128 * j + 16 * k) % N_US
            off = lax.rem(lanes + c, jnp.int32(N_US)) * jnp.int32(VPAD)
            v = idx_v[j, pl.ds(k * 16, 16)]
            row = ((v & ~jnp.int32(VB - 1)) + ((v & (VQ - 1)) << 2)
                   + ((v >> VQ_SHIFT) & 3) + off)
            idx_v[j, pl.ds(k * 16, 16)] = row

    copies = []
    for j in range(NCH):
        copies.append(pltpu.async_copy(
            tab.at[idx_v.at[j]], rows_v.at[pl.ds(j * CHUNK, CHUNK)], sem))
    for c in copies:
        c.wait()

    pltpu.sync_copy(rows_v, out.at[pl.ds(wid * RPW, RPW)])


@functools.cache
def _gather():
    return functools.partial(
        pl.kernel,
        mesh=plsc.VectorSubcoreMesh(core_axis_name="c", subcore_axis_name="s"),
        out_type=jax.ShapeDtypeStruct((ROWS, EMB), jnp.float32),
        scratch_types=[
            pltpu.VMEM((NCH, CHUNK), jnp.int32),
            pltpu.VMEM((RPW, EMB), jnp.float32),
            pltpu.SemaphoreType.DMA,
        ],
        compiler_params=pltpu.CompilerParams(use_tc_tiling_on_sc=False),
    )(_gather_body)


def _linearize_body(tab_ref, out_ref):
    t = tab_ref[0]
    q = VB // 4
    out_ref[0] = jnp.concatenate(
        [jnp.transpose(t[:, c * q:(c + 1) * q]) for c in range(4)], axis=1)


def _linearize(tab_t):
    nvb = pl.cdiv(VOCAB, VB)
    return pl.pallas_call(
        _linearize_body,
        grid=(N_US, nvb),
        in_specs=[pl.BlockSpec((1, EMB, VB), lambda f, j: (f, 0, j))],
        out_specs=pl.BlockSpec((1, VB // 4, 4 * EMB), lambda f, j: (f, j, 0)),
        out_shape=jax.ShapeDtypeStruct((N_US, VPAD // 4, 4 * EMB), jnp.float32),
        compiler_params=pltpu.CompilerParams(
            dimension_semantics=("arbitrary", "arbitrary")),
    )(tab_t)


BT = 256


def _mlp_body(ue_ref, ud_ref, ie_ref, id_ref,
              uW1e_ref, uW1d_ref, ub1_ref, uW2_ref, ub2_ref, uW3_ref, ub3_ref,
              iW1e_ref, iW1d_ref, ib1_ref, iW2_ref, ib2_ref, iW3_ref, ib3_ref,
              out_ref):
    def tower(e, d, W1e, W1d, b1, W2, b2, W3, b3):
        h = jnp.dot(e, W1e, preferred_element_type=jnp.float32)
        h = h + jnp.dot(d, W1d, preferred_element_type=jnp.float32) + b1
        h = jnp.maximum(h, 0.0)
        h = jnp.maximum(jnp.dot(h, W2, preferred_element_type=jnp.float32) + b2, 0.0)
        h = jnp.dot(h, W3, preferred_element_type=jnp.float32) + b3
        norm = jnp.sqrt(jnp.sum(h * h, axis=1, keepdims=True))
        return h / norm

    u = tower(ue_ref[...], ud_ref[...], uW1e_ref[...], uW1d_ref[...],
              ub1_ref[...], uW2_ref[...], ub2_ref[...], uW3_ref[...], ub3_ref[...])
    i = tower(ie_ref[...], id_ref[...], iW1e_ref[...], iW1d_ref[...],
              ib1_ref[...], iW2_ref[...], ib2_ref[...], iW3_ref[...], ib3_ref[...])
    out_ref[...] = jnp.concatenate([u, i], axis=1)


def _mlp(ue, ud, ie, idn, uW1e, uW1d, ub1, uW2, ub2, uW3, ub3,
         iW1e, iW1d, ib1, iW2, ib2, iW3, ib3, interpret=False):
    full = lambda shape: pl.BlockSpec(shape, lambda i: (0, 0))
    return pl.pallas_call(
        _mlp_body,
        grid=(B // BT,),
        in_specs=[
            pl.BlockSpec((BT, N_US * EMB), lambda i: (i, 0)),
            pl.BlockSpec((BT, N_UD), lambda i: (i, 0)),
            pl.BlockSpec((BT, N_IS * EMB), lambda i: (i, 0)),
            pl.BlockSpec((BT, N_ID), lambda i: (i, 0)),
            full(uW1e.shape), full(uW1d.shape), full(ub1.shape),
            full(uW2.shape), full(ub2.shape), full(uW3.shape), full(ub3.shape),
            full(iW1e.shape), full(iW1d.shape), full(ib1.shape),
            full(iW2.shape), full(ib2.shape), full(iW3.shape), full(ib3.shape),
        ],
        out_specs=pl.BlockSpec((BT, 128), lambda i: (i, 0)),
        out_shape=jax.ShapeDtypeStruct((B, 128), jnp.float32),
        compiler_params=pltpu.CompilerParams(
            dimension_semantics=("arbitrary",)),
        interpret=interpret,
    )(ue, ud, ie, idn, uW1e, uW1d, ub1, uW2, ub2, uW3, ub3,
      iW1e, iW1d, ib1, iW2, ib2, iW3, ib3)


def kernel(x, user_tables, item_tables, uW1, ub1, uW2, ub2, uW3, ub3,
           iW1, ib1, iW2, ib2, iW3, ib3):
    us = x[:, :N_US].astype(jnp.int32)
    ud = x[:, N_US:N_US + N_UD]
    it = x[:, N_US + N_UD:N_US + N_UD + N_IS].astype(jnp.int32)
    idn = x[:, N_US + N_UD + N_IS:]

    uidx = us.reshape(NW, NCH, CHUNK)
    iidx = it.reshape(NW, NCH, CHUNK)
    itab_lin = _linearize(jnp.transpose(item_tables, (0, 2, 1)))
    ie_flat = _gather()(itab_lin.reshape(N_IS * VPAD, EMB), iidx)
    utab_lin = _linearize(jnp.transpose(user_tables, (0, 2, 1)))
    ue_flat = _gather()(utab_lin.reshape(N_US * VPAD, EMB), uidx)
    ue = ue_flat.reshape(B, N_US * EMB)
    ie = ie_flat.reshape(B, N_IS * EMB)

    D_E = N_US * EMB
    return _mlp(ue, ud, ie, idn,
                uW1[:D_E], uW1[D_E:], ub1.reshape(1, -1),
                uW2, ub2.reshape(1, -1), uW3, ub3.reshape(1, -1),
                iW1[:D_E], iW1[D_E:], ib1.reshape(1, -1),
                iW2, ib2.reshape(1, -1), iW3, ib3.reshape(1, -1))
